# baseline (speedup 1.0000x reference)
import functools

import jax
import jax.numpy as jnp
from jax import lax
from jax.experimental import pallas as pl
from jax.experimental.pallas import tpu as pltpu
from jax.experimental.pallas import tpu_sc as plsc

INPUT_DIM = 1000000
OUTPUT_DIM = 64
PAD_DIM = 128
BATCH = 4096
SEQ_LEN = 200

NC, NS = 2, 16
NW = NC * NS
CHUNK = BATCH // NW
NBUF = 5

TCHUNK = 256
MAIN_CHUNKS = (INPUT_DIM // TCHUNK // NW) * NW
PER_W_T = MAIN_CHUNKS // NW
EXTRA_BASE = MAIN_CHUNKS * TCHUNK
N_EXTRA = (INPUT_DIM - EXTRA_BASE) // TCHUNK
TAIL_BASE = EXTRA_BASE + N_EXTRA * TCHUNK
TAIL_ROWS = INPUT_DIM - TAIL_BASE

_mesh = plsc.VectorSubcoreMesh(core_axis_name="c", subcore_axis_name="s")


@functools.partial(
    pl.kernel,
    out_type=jax.ShapeDtypeStruct((INPUT_DIM, PAD_DIM), jnp.float32),
    mesh=_mesh,
    scratch_types=[
        pltpu.VMEM((2, OUTPUT_DIM, TCHUNK + 1), jnp.float32),
        pltpu.VMEM((2, TCHUNK, PAD_DIM), jnp.float32),
        pltpu.SemaphoreType.DMA((2,)),
        pltpu.SemaphoreType.DMA((2,)),
    ],
    compiler_params=pltpu.CompilerParams(needs_layout_passes=False),
)
def _relayout(wt_hbm, wtail_hbm, w128_hbm, tin_v, tout_v, in_sems, out_sems):
    wid = lax.axis_index("s") * NC + lax.axis_index("c")

    rows16 = [jnp.arange(16, dtype=jnp.int32) + 16 * g for g in range(4)]

    def start_in(c0, j):
        pltpu.async_copy(
            wt_hbm.at[:, pl.ds(c0, TCHUNK)],
            tin_v.at[j, :, pl.ds(0, TCHUNK)],
            in_sems.at[j],
        )

    def wait_in(c0, j):
        pltpu.make_async_copy(
            wt_hbm.at[:, pl.ds(c0, TCHUNK)],
            tin_v.at[j, :, pl.ds(0, TCHUNK)],
            in_sems.at[j],
        ).wait()

    def transpose(j):
        @plsc.parallel_loop(
            0, TCHUNK, unroll=16, carry=jnp.zeros((16,), jnp.int32)
        )
        def _rows(k, cols):
            for g in range(4):
                vals = plsc.load_gather(tin_v.at[j], [rows16[g], cols])
                tout_v[j, k, pl.ds(16 * g, 16)] = vals
            return cols + 1

    def start_write(c0, j):
        pltpu.async_copy(
            tout_v.at[j], w128_hbm.at[pl.ds(c0, TCHUNK)], out_sems.at[j]
        )

    def wait_write(c0, j):
        pltpu.make_async_copy(
            tout_v.at[j], w128_hbm.at[pl.ds(c0, TCHUNK)], out_sems.at[j]
        ).wait()

    base = wid * PER_W_T * TCHUNK

    for j in range(2):
        start_in(base + j * TCHUNK, j)

    for j in range(2):
        c0 = base + j * TCHUNK
        wait_in(c0, j)
        transpose(j)
        start_write(c0, j)
        start_in(c0 + 2 * TCHUNK, j)

    @pl.loop(2, PER_W_T - 2, step=2)
    def _chunks(t0):
        for j in range(2):
            c0 = base + (t0 + j) * TCHUNK
            wait_in(c0, j)
            wait_write(c0, j)
            transpose(j)
            start_write(c0, j)
            start_in(c0 + 2 * TCHUNK, j)

    for j in range(2):
        c0 = base + (PER_W_T - 2 + j) * TCHUNK
        wait_in(c0, j)
        wait_write(c0, j)
        transpose(j)
        start_write(c0, j)
    for j in range(2):
        wait_write(0, j)

    @pl.when(wid < N_EXTRA)
    def _extra():
        c0 = EXTRA_BASE + wid * TCHUNK
        start_in(c0, 0)
        wait_in(c0, 0)
        transpose(0)
        start_write(c0, 0)
        wait_write(c0, 0)

    @pl.when(wid == N_EXTRA)
    def _tail():
        pltpu.sync_copy(wtail_hbm, w128_hbm.at[pl.ds(TAIL_BASE, TAIL_ROWS)])


@functools.partial(
    pl.kernel,
    out_type=jax.ShapeDtypeStruct((BATCH, SEQ_LEN, PAD_DIM), jnp.float32),
    mesh=_mesh,
    scratch_types=[
        pltpu.VMEM((SEQ_LEN, CHUNK), jnp.int32),
        pltpu.VMEM((NBUF, CHUNK, PAD_DIM), jnp.float32),
        pltpu.SemaphoreType.DMA((NBUF,)),
        pltpu.SemaphoreType.DMA((NBUF,)),
    ],
)
def _emb_lookup(xt_hbm, w_hbm, out_hbm, idx_v, rows_v, in_sems, out_sems):
    wid = lax.axis_index("s") * NC + lax.axis_index("c")
    b0 = wid * CHUNK

    pltpu.sync_copy(xt_hbm.at[:, pl.ds(b0, CHUNK)], idx_v)

    def start_gather(s, j):
        pltpu.async_copy(w_hbm.at[idx_v.at[s]], rows_v.at[j], in_sems.at[j])

    def wait_gather(s, j):
        pltpu.make_async_copy(
            w_hbm.at[idx_v.at[s]], rows_v.at[j], in_sems.at[j]
        ).wait()

    def start_write(s, j):
        pltpu.async_copy(
            rows_v.at[j], out_hbm.at[pl.ds(b0, CHUNK), s], out_sems.at[j]
        )

    def wait_write(s, j):
        pltpu.make_async_copy(
            rows_v.at[j], out_hbm.at[pl.ds(b0, CHUNK), s], out_sems.at[j]
        ).wait()

    for j in range(NBUF):
        start_gather(j, j)

    @pl.loop(0, SEQ_LEN - NBUF, step=NBUF)
    def _ring(s0):
        for j in range(NBUF):
            s = s0 + j
            wait_gather(s, j)
            start_write(s, j)
            wait_write(s, j)
            start_gather(s + NBUF, j)

    for j in range(NBUF):
        s = SEQ_LEN - NBUF + j
        wait_gather(s, j)
        start_write(s, j)
    for j in range(NBUF):
        s = SEQ_LEN - NBUF + j
        wait_write(s, j)


def kernel(x, W):
    xt = x.astype(jnp.int32).T
    wt = W.T
    wtail = jnp.pad(
        W[TAIL_BASE:, :], ((0, 0), (0, PAD_DIM - OUTPUT_DIM))
    )
    w128 = _relayout(wt, wtail)
    out = _emb_lookup(xt, w128)
    return out[..., :OUTPUT_DIM]


# speedup vs baseline: 1.2726x; 1.2726x over previous
import functools
import jax
import jax.numpy as jnp
from jax import lax
from jax.experimental import pallas as pl
from jax.experimental.pallas import tpu as pltpu
from jax.experimental.pallas import tpu_sc as plsc

INPUT_DIM = 1000000
OUTPUT_DIM = 64
PAD_DIM = 128
BATCH = 4096
SEQ_LEN = 200
NC, NS = 2, 16
NW = NC * NS
CHUNK = BATCH // NW
NBUF = 5

_mesh = plsc.VectorSubcoreMesh(core_axis_name="c", subcore_axis_name="s")

@functools.partial(
    pl.kernel,
    out_type=jax.ShapeDtypeStruct((BATCH, SEQ_LEN, PAD_DIM), jnp.float32),
    mesh=_mesh,
    scratch_types=[
        pltpu.VMEM((SEQ_LEN, CHUNK), jnp.int32),
        pltpu.VMEM((NBUF, CHUNK, PAD_DIM), jnp.float32),
        pltpu.SemaphoreType.DMA((NBUF,)),
        pltpu.SemaphoreType.DMA((NBUF,)),
    ],
    compiler_params=pltpu.CompilerParams(needs_layout_passes=False),
)
def _emb_lookup(xt_hbm, w_hbm, out_hbm, idx_v, rows_v, in_sems, out_sems):
    wid = lax.axis_index("s") * NC + lax.axis_index("c")
    b0 = wid * CHUNK
    pltpu.sync_copy(xt_hbm.at[:, pl.ds(b0, CHUNK)], idx_v)

    def start_gather(s, j):
        pltpu.async_copy(w_hbm.at[idx_v.at[s]], rows_v.at[j], in_sems.at[j])

    def wait_gather(s, j):
        pltpu.make_async_copy(w_hbm.at[idx_v.at[s]], rows_v.at[j], in_sems.at[j]).wait()

    def start_write(s, j):
        pltpu.async_copy(rows_v.at[j], out_hbm.at[pl.ds(b0, CHUNK), s], out_sems.at[j])

    def wait_write(s, j):
        pltpu.make_async_copy(rows_v.at[j], out_hbm.at[pl.ds(b0, CHUNK), s], out_sems.at[j]).wait()

    for j in range(NBUF):
        start_gather(j, j)

    @pl.loop(0, SEQ_LEN - NBUF, step=NBUF)
    def _ring(s0):
        for j in range(NBUF):
            s = s0 + j
            wait_gather(s, j)
            start_write(s, j)
            wait_write(s, j)
            start_gather(s + NBUF, j)

    for j in range(NBUF):
        s = SEQ_LEN - NBUF + j
        wait_gather(s, j)
        start_write(s, j)
    for j in range(NBUF):
        s = SEQ_LEN - NBUF + j
        wait_write(s, j)


def kernel(x, W):
    xt = x.astype(jnp.int32).T
    w_pad = jnp.pad(W, ((0, 0), (0, PAD_DIM - OUTPUT_DIM)))
    out = _emb_lookup(xt, w_pad)
    return out[..., :OUTPUT_DIM]
